# baseline (speedup 1.0000x reference)
import functools

import jax
import jax.numpy as jnp
from jax.experimental import pallas as pl
from jax.experimental.pallas import tpu as pltpu
from jax.experimental.pallas import tpu_sc as plsc

NG = 48
NNGB = 12
DIM = 3
NSHELL = 6
NCH0 = 4

_GATHER_WINDOW = 128
_NS_PAD = 102400
_SITE_BLOCK = 2048


def _sc_gather(table, idx):
    n = idx.shape[0]
    d = table.shape[1]
    idx2 = idx.reshape(1, n)
    mesh = plsc.VectorSubcoreMesh(core_axis_name="core",
                                  subcore_axis_name="subcore")

    @functools.partial(
        pl.kernel,
        out_type=jax.ShapeDtypeStruct((n, d), jnp.float32),
        mesh=mesh,
        compiler_params=pltpu.CompilerParams(use_tc_tiling_on_sc=False),
    )
    def k(tab_hbm, i_hbm, o_hbm):
        def body(i_vmem, o_vmem):
            pltpu.sync_copy(tab_hbm.at[i_vmem.at[0]], o_vmem)

        pltpu.emit_pipeline(
            body,
            grid=(n // _GATHER_WINDOW,),
            in_specs=[pl.BlockSpec((1, _GATHER_WINDOW),
                                   index_map=lambda i: (0, i))],
            out_specs=[pl.BlockSpec((_GATHER_WINDOW, d),
                                    index_map=lambda i: (i, 0))],
            core_axis_name=("core", "subcore"),
            dimension_semantics=(pltpu.PARALLEL,),
        )(i_hbm, o_hbm)

    return k(table, idx2)


def _softplus(x):
    return jnp.maximum(x, 0.0) + jnp.log(1.0 + jnp.exp(-jnp.abs(x)))


def _tc_layer(g, w, b, nch_out):
    k_dim, nb, ns = g.shape
    s = _SITE_BLOCK

    def body(g_ref, w_ref, b_ref, o_ref):
        wv = w_ref[...]
        bv = b_ref[...]
        for bb in range(nb):
            x = g_ref[:, bb, :]
            h = jnp.dot(wv, x, preferred_element_type=jnp.float32) + bv
            h = _softplus(h)
            h = h.reshape(nch_out, NG, s).sum(axis=1) * (1.0 / NG)
            o_ref[:, bb, :] = h

    return pl.pallas_call(
        body,
        grid=(ns // s,),
        in_specs=[
            pl.BlockSpec((k_dim, nb, s), lambda i: (0, 0, i)),
            pl.BlockSpec(w.shape, lambda i: (0, 0)),
            pl.BlockSpec(b.shape, lambda i: (0, 0)),
        ],
        out_specs=pl.BlockSpec((nch_out, nb, s), lambda i: (0, 0, i)),
        out_shape=jax.ShapeDtypeStruct((nch_out, nb, ns), jnp.float32),
    )(g, w, b)


def _tc_final(g, w2, shells, shell_w):
    _, nb, ns = g.shape
    s = _SITE_BLOCK

    def body(g_ref, w_ref, sh_ref, sw_ref, o_ref):
        i = pl.program_id(0)
        sh = sh_ref[...].reshape(1, 1, s)
        swtab = sw_ref[...]
        sw = jnp.zeros((1, 1, s), jnp.float32)
        for q in range(NSHELL):
            sw = jnp.where(sh == q, swtab[0, q], sw)
        gw = jnp.sum(g_ref[...] * sw, axis=2)
        part = jnp.dot(w_ref[...], gw, preferred_element_type=jnp.float32)

        @pl.when(i == 0)
        def _():
            o_ref[...] = jnp.zeros_like(o_ref)

        o_ref[...] += part

    return pl.pallas_call(
        body,
        grid=(ns // s,),
        in_specs=[
            pl.BlockSpec((NNGB, nb, s), lambda i: (0, 0, i)),
            pl.BlockSpec(w2.shape, lambda i: (0, 0)),
            pl.BlockSpec((1, s), lambda i: (0, i)),
            pl.BlockSpec((1, NSHELL), lambda i: (0, 0)),
        ],
        out_specs=pl.BlockSpec((DIM, nb), lambda i: (0, 0)),
        out_shape=jax.ShapeDtypeStruct((DIM, nb), jnp.float32),
    )(g, w2, shells.reshape(1, ns), shell_w.reshape(1, NSHELL))


def _rotate_weight(weight, bias, perms):
    nch_out, nch_in, _ = weight.shape
    wrep = jnp.repeat(weight, NG, axis=0)
    perm = jnp.tile(perms, (nch_out, nch_in)).reshape(-1, nch_in, NNGB)
    wperm = jnp.take_along_axis(wrep, perm, axis=2).reshape(-1, nch_in * NNGB)
    brep = jnp.repeat(bias, NG, axis=0)
    return wperm, brep


def kernel(InStates, GnnPerms, gdiags, NNSites, SitesToShells, Psi_0, bias_0,
           Psi_1, bias_1, wtVC, ShellWeights):
    nb, _, ns = InStates.shape

    gw0, gb0 = _rotate_weight(Psi_0, bias_0, GnnPerms)
    gw1, gb1 = _rotate_weight(Psi_1, bias_1, GnnPerms)
    wtvc_rep = jnp.tile(wtVC, (NG, 1))
    perm2 = jnp.repeat(GnnPerms, DIM, axis=0)
    wtvc_transf = jnp.matmul(gdiags, jnp.take_along_axis(wtvc_rep, perm2,
                                                         axis=1))
    w2 = wtvc_transf.reshape(NG, DIM, NNGB).mean(axis=0)

    nsp = _NS_PAD
    idx = jnp.pad(NNSites.astype(jnp.int32).T,
                  ((0, nsp - ns), (0, 0))).reshape(-1)
    shells_p = jnp.pad(SitesToShells.astype(jnp.int32), (0, nsp - ns),
                       constant_values=NSHELL)

    x_t = InStates[:, 0, :].T
    g0 = _sc_gather(x_t, idx)
    a_in = g0.reshape(nsp, NNGB, nb).transpose(1, 2, 0)
    h0 = _tc_layer(a_in, gw0, gb0, NCH0)

    t1 = h0.transpose(2, 0, 1).reshape(nsp, NCH0 * nb)
    g1 = _sc_gather(t1, idx)
    b_in = (g1.reshape(nsp, NNGB, NCH0, nb)
              .transpose(2, 1, 3, 0)
              .reshape(NCH0 * NNGB, nb, nsp))
    h1 = _tc_layer(b_in, gw1, gb1, 1)

    t2 = h1[0].T
    g2 = _sc_gather(t2, idx)
    c_in = g2.reshape(nsp, NNGB, nb).transpose(1, 2, 0)
    z = _tc_final(c_in, w2, shells_p, ShellWeights)
    return z.T * (1.0 / ns)


# speedup vs baseline: 2.7751x; 2.7751x over previous
import functools

import jax
import jax.numpy as jnp
from jax import lax
from jax.experimental import pallas as pl
from jax.experimental.pallas import tpu as pltpu
from jax.experimental.pallas import tpu_sc as plsc

NG = 48
NNGB = 12
DIM = 3
NSHELL = 6
NCH0 = 4
NB = 8

_NSP = 102400
_S = 2048
_NW = 32

_SC_PARAMS = pltpu.CompilerParams(use_tc_tiling_on_sc=False,
                                  needs_layout_passes=False)


def _mesh():
    return plsc.VectorSubcoreMesh(core_axis_name="core",
                                  subcore_axis_name="subcore")


def _wid():
    return lax.axis_index("subcore") * 2 + lax.axis_index("core")


def _iota16():
    return lax.iota(jnp.int32, 16)


def _sc_table(x, d, k):
    nsp = x.shape[1]
    nq = nsp // k

    @functools.partial(
        pl.kernel,
        out_type=jax.ShapeDtypeStruct((nsp, d), jnp.float32),
        mesh=_mesh(),
        scratch_types=[
            pltpu.VMEM((d, k), jnp.float32),
            pltpu.VMEM((k, d), jnp.float32),
            pltpu.SemaphoreType.DMA,
        ],
        compiler_params=_SC_PARAMS,
    )
    def kk(x_hbm, t_hbm, bufT, buf, sem):
        w = _wid()
        iot = _iota16()

        @pl.loop(0, (nq + _NW - 1) // _NW)
        def _(it):
            q = w + it * _NW

            @pl.when(q < nq)
            def _():
                s0 = q * k
                pltpu.async_copy(x_hbm.at[:, pl.ds(s0, k)], bufT, sem).wait()

                @pl.loop(0, k // 16)
                def _(g):
                    rows = g * 16 + iot
                    for m in range(d):
                        v = bufT[m, pl.ds(g * 16, 16)]
                        plsc.store_scatter(
                            buf, [rows, jnp.full((16,), m, jnp.int32)], v)

                pltpu.async_copy(buf, t_hbm.at[pl.ds(s0, k), :], sem).wait()

    return kk(x)


def _sc_gather_t(table, idx2d, k):
    nsp, d = table.shape
    nq = nsp // k
    sub = k // 128

    @functools.partial(
        pl.kernel,
        out_type=jax.ShapeDtypeStruct((d, NNGB, nsp), jnp.float32),
        mesh=_mesh(),
        scratch_types=[
            pltpu.VMEM((k,), jnp.int32),
            pltpu.VMEM((k, d), jnp.float32),
            pltpu.VMEM((d, k), jnp.float32),
            pltpu.SemaphoreType.DMA,
            pltpu.SemaphoreType.DMA,
        ],
        compiler_params=_SC_PARAMS,
    )
    def kk(t_hbm, i_hbm, o_hbm, idxv, buf, bufT, sem, sem2):
        w = _wid()
        iot = _iota16()

        for j in range(NNGB):
            @pl.loop(0, (nq + _NW - 1) // _NW)
            def _(it):
                q = w + it * _NW

                @pl.when(q < nq)
                def _():
                    s0 = q * k
                    pltpu.async_copy(i_hbm.at[j, pl.ds(s0, k)], idxv,
                                     sem).wait()
                    cps = [
                        pltpu.async_copy(
                            t_hbm.at[idxv.at[pl.ds(g * 128, 128)]],
                            buf.at[pl.ds(g * 128, 128), :], sem2)
                        for g in range(sub)
                    ]
                    for c in cps:
                        c.wait()

                    @pl.loop(0, k // 16)
                    def _(g):
                        rows = g * 16 + iot
                        for m in range(d):
                            v = plsc.load_gather(
                                buf, [rows, jnp.full((16,), m, jnp.int32)])
                            bufT[m, pl.ds(g * 16, 16)] = v

                    ocs = [
                        pltpu.async_copy(bufT.at[m],
                                         o_hbm.at[m, j, pl.ds(s0, k)], sem)
                        for m in range(d)
                    ]
                    for c in ocs:
                        c.wait()

    return kk(table, idx2d)


def _sc_hist(idx2d, sw_site, k):
    nsp = idx2d.shape[1]
    nq = nsp // k
    sub = k // 128
    zk = 1600
    per_tile = nsp // 16

    @functools.partial(
        pl.kernel,
        out_type=jax.ShapeDtypeStruct((2, NNGB, nsp), jnp.float32),
        mesh=_mesh(),
        scratch_types=[
            pltpu.VMEM((k,), jnp.int32),
            pltpu.VMEM((k,), jnp.float32),
            pltpu.VMEM((zk,), jnp.float32),
            pltpu.SemaphoreType.DMA,
        ] + [pltpu.VMEM_SHARED((nsp,), jnp.float32) for _ in range(NNGB)],
        compiler_params=_SC_PARAMS,
    )
    def kk(i_hbm, w_hbm, o_hbm, idxv, swv, zbuf, sem, *sh):
        cid = lax.axis_index("core")
        sid = lax.axis_index("subcore")
        w = _wid()

        @pl.loop(0, zk // 16)
        def _(z):
            zbuf[pl.ds(z * 16, 16)] = jnp.zeros((16,), jnp.float32)

        for j in range(NNGB):
            for r in range(per_tile // zk):
                pltpu.async_copy(
                    zbuf, sh[j].at[pl.ds(sid * per_tile + r * zk, zk)],
                    sem).wait()
        plsc.subcore_barrier()

        for j in range(NNGB):
            @pl.loop(0, (nq + _NW - 1) // _NW)
            def _(it):
                q = w + it * _NW

                @pl.when(q < nq)
                def _():
                    s0 = q * k
                    pltpu.async_copy(i_hbm.at[j, pl.ds(s0, k)], idxv,
                                     sem).wait()
                    pltpu.async_copy(w_hbm.at[pl.ds(s0, k)], swv, sem).wait()
                    cps = [
                        pltpu.async_copy(
                            swv.at[pl.ds(g * 128, 128)],
                            sh[j].at[idxv.at[pl.ds(g * 128, 128)]],
                            sem, add=True)
                        for g in range(sub)
                    ]
                    for c in cps:
                        c.wait()

        plsc.subcore_barrier()
        for j in range(NNGB):
            pltpu.async_copy(
                sh[j].at[pl.ds(sid * per_tile, per_tile)],
                o_hbm.at[cid, j, pl.ds(sid * per_tile, per_tile)],
                sem).wait()

    return kk(idx2d, sw_site)


def _softplus(x):
    return jnp.maximum(x, 0.0) + jnp.log(1.0 + jnp.exp(-jnp.abs(x)))


def _tc_layer_a(a, w, b):
    _, _, nsp = a.shape

    def body(a_ref, w_ref, b_ref, o_ref):
        wv = w_ref[...]
        bv = b_ref[...]
        for bb in range(NB):
            x = a_ref[bb]
            h = jnp.dot(wv, x, preferred_element_type=jnp.float32) + bv
            h = _softplus(h)
            h = h.reshape(NCH0, NG, _S).sum(axis=1) * (1.0 / NG)
            o_ref[4 * bb:4 * bb + 4, :] = h

    return pl.pallas_call(
        body,
        grid=(nsp // _S,),
        in_specs=[
            pl.BlockSpec((NB, NNGB, _S), lambda i: (0, 0, i)),
            pl.BlockSpec(w.shape, lambda i: (0, 0)),
            pl.BlockSpec(b.shape, lambda i: (0, 0)),
        ],
        out_specs=pl.BlockSpec((NCH0 * NB, _S), lambda i: (0, i)),
        out_shape=jax.ShapeDtypeStruct((NCH0 * NB, nsp), jnp.float32),
    )(a, w, b)


def _tc_layer_b(bgat, w, b):
    _, _, nsp = bgat.shape

    def body(g_ref, w_ref, b_ref, o_ref):
        wv = w_ref[...]
        bv = b_ref[...]
        for bb in range(NB):
            x = g_ref[4 * bb:4 * bb + 4].reshape(NCH0 * NNGB, _S)
            h = jnp.dot(wv, x, preferred_element_type=jnp.float32) + bv
            h = _softplus(h)
            o_ref[bb, :] = h.sum(axis=0) * (1.0 / NG)

    return pl.pallas_call(
        body,
        grid=(nsp // _S,),
        in_specs=[
            pl.BlockSpec((NCH0 * NB, NNGB, _S), lambda i: (0, 0, i)),
            pl.BlockSpec(w.shape, lambda i: (0, 0)),
            pl.BlockSpec(b.shape, lambda i: (0, 0)),
        ],
        out_specs=pl.BlockSpec((NB, _S), lambda i: (0, i)),
        out_shape=jax.ShapeDtypeStruct((NB, nsp), jnp.float32),
    )(bgat, w, b)


def _tc_final(h1, dp, w2):
    _, nsp = h1.shape

    def body(h_ref, d_ref, w_ref, o_ref):
        i = pl.program_id(0)
        dsum = d_ref[0] + d_ref[1]
        c3 = jnp.dot(w_ref[...], dsum,
                     preferred_element_type=jnp.float32)
        p = lax.dot_general(h_ref[...], c3, (((1,), (1,)), ((), ())),
                            preferred_element_type=jnp.float32)

        @pl.when(i == 0)
        def _():
            o_ref[...] = jnp.zeros_like(o_ref)

        o_ref[...] += p

    return pl.pallas_call(
        body,
        grid=(nsp // _S,),
        in_specs=[
            pl.BlockSpec((NB, _S), lambda i: (0, i)),
            pl.BlockSpec((2, NNGB, _S), lambda i: (0, 0, i)),
            pl.BlockSpec(w2.shape, lambda i: (0, 0)),
        ],
        out_specs=pl.BlockSpec((NB, DIM), lambda i: (0, 0)),
        out_shape=jax.ShapeDtypeStruct((NB, DIM), jnp.float32),
    )(h1, dp, w2)


def _rotate_weight(weight, bias, perms):
    nch_out, nch_in, _ = weight.shape
    wrep = jnp.repeat(weight, NG, axis=0)
    perm = jnp.tile(perms, (nch_out, nch_in)).reshape(-1, nch_in, NNGB)
    wperm = jnp.take_along_axis(wrep, perm, axis=2).reshape(-1, nch_in * NNGB)
    brep = jnp.repeat(bias, NG, axis=0)
    return wperm, brep


def kernel(InStates, GnnPerms, gdiags, NNSites, SitesToShells, Psi_0, bias_0,
           Psi_1, bias_1, wtVC, ShellWeights):
    nb, _, ns = InStates.shape
    nsp = _NSP

    gw0, gb0 = _rotate_weight(Psi_0, bias_0, GnnPerms)
    gw1, gb1 = _rotate_weight(Psi_1, bias_1, GnnPerms)
    wtvc_rep = jnp.tile(wtVC, (NG, 1))
    perm2 = jnp.repeat(GnnPerms, DIM, axis=0)
    wtvc_transf = jnp.matmul(gdiags, jnp.take_along_axis(wtvc_rep, perm2,
                                                         axis=1))
    w2 = wtvc_transf.reshape(NG, DIM, NNGB).mean(axis=0)

    idx2d = jnp.pad(NNSites.astype(jnp.int32), ((0, 0), (0, nsp - ns)))
    x0p = jnp.pad(InStates[:, 0, :], ((0, 0), (0, nsp - ns)))
    sw = jnp.zeros((ns,), jnp.float32)
    for q in range(NSHELL):
        sw = jnp.where(SitesToShells == q, ShellWeights[q], sw)
    sw_site = jnp.pad(sw, (0, nsp - ns))

    t0 = _sc_table(x0p, NB, 2048)
    a = _sc_gather_t(t0, idx2d, 2048)
    h0 = _tc_layer_a(a, gw0, gb0)

    t1 = _sc_table(h0, NCH0 * NB, 1024)
    bgat = _sc_gather_t(t1, idx2d, 1024)
    h1 = _tc_layer_b(bgat, gw1, gb1)

    dp = _sc_hist(idx2d, sw_site, 2048)
    out = _tc_final(h1, dp, w2)
    return out * (1.0 / ns)


# speedup vs baseline: 3.8435x; 1.3850x over previous
import functools

import jax
import jax.numpy as jnp
from jax import lax
from jax.experimental import pallas as pl
from jax.experimental.pallas import tpu as pltpu
from jax.experimental.pallas import tpu_sc as plsc

NG = 48
NNGB = 12
DIM = 3
NSHELL = 6
NCH0 = 4
NB = 8

_NSP = 102400
_S = 2048
_NW = 32

_SC_PARAMS = pltpu.CompilerParams(use_tc_tiling_on_sc=False,
                                  needs_layout_passes=False)


def _mesh():
    return plsc.VectorSubcoreMesh(core_axis_name="core",
                                  subcore_axis_name="subcore")


def _wid():
    return lax.axis_index("subcore") * 2 + lax.axis_index("core")


def _iota16():
    return lax.iota(jnp.int32, 16)


def _sc_table(x, d, k):
    nsp = x.shape[1]
    nq = nsp // k

    @functools.partial(
        pl.kernel,
        out_type=jax.ShapeDtypeStruct((nsp, d), jnp.float32),
        mesh=_mesh(),
        scratch_types=[
            pltpu.VMEM((d, k), jnp.float32),
            pltpu.VMEM((k, d), jnp.float32),
            pltpu.SemaphoreType.DMA,
        ],
        compiler_params=_SC_PARAMS,
    )
    def kk(x_hbm, t_hbm, bufT, buf, sem):
        w = _wid()
        iot = _iota16()

        @pl.loop(0, (nq + _NW - 1) // _NW)
        def _(it):
            q = w + it * _NW

            @pl.when(q < nq)
            def _():
                s0 = q * k
                pltpu.async_copy(x_hbm.at[:, pl.ds(s0, k)], bufT, sem).wait()

                @pl.loop(0, k // 16)
                def _(g):
                    rows = g * 16 + iot
                    for m in range(d):
                        v = bufT[m, pl.ds(g * 16, 16)]
                        plsc.store_scatter(
                            buf, [rows, jnp.full((16,), m, jnp.int32)], v)

                pltpu.async_copy(buf, t_hbm.at[pl.ds(s0, k), :], sem).wait()

    return kk(x)


def _sc_gather_t(table, idx2d, k):
    nsp, d = table.shape
    nq = nsp // k
    nt = nq // _NW
    assert nt * _NW == nq
    sub = k // 128

    @functools.partial(
        pl.kernel,
        out_type=jax.ShapeDtypeStruct((NNGB, nq, d, k), jnp.float32),
        mesh=_mesh(),
        scratch_types=[
            pltpu.VMEM((2, k), jnp.int32),
            pltpu.VMEM((2, k, d), jnp.float32),
            pltpu.VMEM((2, d, k), jnp.float32),
            pltpu.SemaphoreType.DMA,
            pltpu.SemaphoreType.DMA,
            pltpu.SemaphoreType.DMA,
        ],
        compiler_params=_SC_PARAMS,
    )
    def kk(t_hbm, i_hbm, o_hbm, idxv, buf, bufT, semI, semG, semO):
        w = _wid()
        iot = _iota16()

        def fire_idx(j, t, p):
            q = w + t * _NW
            return pltpu.async_copy(i_hbm.at[j, pl.ds(q * k, k)],
                                    idxv.at[p], semI)

        def fire_gathers(p):
            return [
                pltpu.async_copy(
                    t_hbm.at[idxv.at[p].at[pl.ds(g * 128, 128)]],
                    buf.at[p].at[pl.ds(g * 128, 128), :], semG)
                for g in range(sub)
            ]

        def transpose(p):
            @pl.loop(0, k // 16)
            def _(g):
                rows = g * 16 + iot
                for m in range(d):
                    v = plsc.load_gather(
                        buf.at[p], [rows, jnp.full((16,), m, jnp.int32)])
                    bufT[p, m, pl.ds(g * 16, 16)] = v

        def fire_out(j, t, p):
            q = w + t * _NW
            return pltpu.async_copy(bufT.at[p], o_hbm.at[j, q], semO)

        @pl.loop(0, NNGB)
        def _(j):
            dI = {0: fire_idx(j, 0, 0)}
            dG = {}
            dO = {}
            for t in range(nt):
                p = t % 2
                dI[t].wait()
                dG[t] = fire_gathers(p)
                if t == 0:
                    if nt > 1:
                        dI[1] = fire_idx(j, 1, 1)
                else:
                    for c in dG[t - 1]:
                        c.wait()
                    if t + 1 < nt:
                        dI[t + 1] = fire_idx(j, t + 1, 1 - p)
                    if t >= 3:
                        dO[t - 3].wait()
                    transpose(1 - p)
                    dO[t - 1] = fire_out(j, t - 1, 1 - p)
            pl_ = (nt - 1) % 2
            for c in dG[nt - 1]:
                c.wait()
            if nt >= 3:
                dO[nt - 3].wait()
            transpose(pl_)
            dO[nt - 1] = fire_out(j, nt - 1, pl_)
            if nt >= 2:
                dO[nt - 2].wait()
            dO[nt - 1].wait()

    return kk(table, idx2d)


def _sc_hist(idx2d, sw_site, k):
    nsp = idx2d.shape[1]
    nq = nsp // k
    sub = k // 128
    zk = 1600
    per_tile = nsp // 16

    @functools.partial(
        pl.kernel,
        out_type=jax.ShapeDtypeStruct((2, NNGB, nsp), jnp.float32),
        mesh=_mesh(),
        scratch_types=[
            pltpu.VMEM((k,), jnp.int32),
            pltpu.VMEM((k,), jnp.float32),
            pltpu.VMEM((zk,), jnp.float32),
            pltpu.SemaphoreType.DMA,
        ] + [pltpu.VMEM_SHARED((nsp,), jnp.float32) for _ in range(NNGB)],
        compiler_params=_SC_PARAMS,
    )
    def kk(i_hbm, w_hbm, o_hbm, idxv, swv, zbuf, sem, *sh):
        cid = lax.axis_index("core")
        sid = lax.axis_index("subcore")
        w = _wid()

        @pl.loop(0, zk // 16)
        def _(z):
            zbuf[pl.ds(z * 16, 16)] = jnp.zeros((16,), jnp.float32)

        for j in range(NNGB):
            for r in range(per_tile // zk):
                pltpu.async_copy(
                    zbuf, sh[j].at[pl.ds(sid * per_tile + r * zk, zk)],
                    sem).wait()
        plsc.subcore_barrier()

        for j in range(NNGB):
            @pl.loop(0, (nq + _NW - 1) // _NW)
            def _(it):
                q = w + it * _NW

                @pl.when(q < nq)
                def _():
                    s0 = q * k
                    pltpu.async_copy(i_hbm.at[j, pl.ds(s0, k)], idxv,
                                     sem).wait()
                    pltpu.async_copy(w_hbm.at[pl.ds(s0, k)], swv, sem).wait()
                    cps = [
                        pltpu.async_copy(
                            swv.at[pl.ds(g * 128, 128)],
                            sh[j].at[idxv.at[pl.ds(g * 128, 128)]],
                            sem, add=True)
                        for g in range(sub)
                    ]
                    for c in cps:
                        c.wait()

        plsc.subcore_barrier()
        for j in range(NNGB):
            pltpu.async_copy(
                sh[j].at[pl.ds(sid * per_tile, per_tile)],
                o_hbm.at[cid, j, pl.ds(sid * per_tile, per_tile)],
                sem).wait()

    return kk(idx2d, sw_site)


def _softplus(x):
    return jnp.maximum(x, 0.0) + jnp.log(1.0 + jnp.exp(-jnp.abs(x)))


def _tc_layer_a(a, w, b, k, cpb):
    _, nq, _, _ = a.shape
    nsp = nq * k

    def body(a_ref, w_ref, b_ref, o_ref):
        wv = w_ref[...]
        bv = b_ref[...]
        for cc in range(cpb):
            for bb in range(NB):
                x = a_ref[:, cc, bb, :]
                h = jnp.dot(wv, x, preferred_element_type=jnp.float32) + bv
                h = _softplus(h)
                h = h.reshape(NCH0, NG, k).sum(axis=1) * (1.0 / NG)
                o_ref[4 * bb:4 * bb + 4, pl.ds(cc * k, k)] = h

    return pl.pallas_call(
        body,
        grid=(nq // cpb,),
        in_specs=[
            pl.BlockSpec((NNGB, cpb, NB, k), lambda i: (0, i, 0, 0)),
            pl.BlockSpec(w.shape, lambda i: (0, 0)),
            pl.BlockSpec(b.shape, lambda i: (0, 0)),
        ],
        out_specs=pl.BlockSpec((NCH0 * NB, cpb * k), lambda i: (0, i)),
        out_shape=jax.ShapeDtypeStruct((NCH0 * NB, nsp), jnp.float32),
    )(a, w, b)


def _tc_layer_b(bgat, w, b, k, cpb):
    _, nq, _, _ = bgat.shape
    nsp = nq * k

    def body(g_ref, w_ref, b_ref, o_ref):
        wv = w_ref[...]
        bv = b_ref[...]
        for cc in range(cpb):
            for bb in range(NB):
                h = bv
                for c in range(NCH0):
                    x = g_ref[:, cc, 4 * bb + c, :]
                    h = h + jnp.dot(wv[:, c * NNGB:(c + 1) * NNGB], x,
                                    preferred_element_type=jnp.float32)
                h = _softplus(h)
                o_ref[bb, pl.ds(cc * k, k)] = h.sum(axis=0) * (1.0 / NG)

    return pl.pallas_call(
        body,
        grid=(nq // cpb,),
        in_specs=[
            pl.BlockSpec((NNGB, cpb, NCH0 * NB, k), lambda i: (0, i, 0, 0)),
            pl.BlockSpec(w.shape, lambda i: (0, 0)),
            pl.BlockSpec(b.shape, lambda i: (0, 0)),
        ],
        out_specs=pl.BlockSpec((NB, cpb * k), lambda i: (0, i)),
        out_shape=jax.ShapeDtypeStruct((NB, nsp), jnp.float32),
    )(bgat, w, b)


def _tc_final(h1, dp, w2):
    _, nsp = h1.shape

    def body(h_ref, d_ref, w_ref, o_ref):
        i = pl.program_id(0)
        dsum = d_ref[0] + d_ref[1]
        c3 = jnp.dot(w_ref[...], dsum,
                     preferred_element_type=jnp.float32)
        p = lax.dot_general(h_ref[...], c3, (((1,), (1,)), ((), ())),
                            preferred_element_type=jnp.float32)

        @pl.when(i == 0)
        def _():
            o_ref[...] = jnp.zeros_like(o_ref)

        o_ref[...] += p

    return pl.pallas_call(
        body,
        grid=(nsp // _S,),
        in_specs=[
            pl.BlockSpec((NB, _S), lambda i: (0, i)),
            pl.BlockSpec((2, NNGB, _S), lambda i: (0, 0, i)),
            pl.BlockSpec(w2.shape, lambda i: (0, 0)),
        ],
        out_specs=pl.BlockSpec((NB, DIM), lambda i: (0, 0)),
        out_shape=jax.ShapeDtypeStruct((NB, DIM), jnp.float32),
    )(h1, dp, w2)


def _rotate_weight(weight, bias, perms):
    nch_out, nch_in, _ = weight.shape
    wrep = jnp.repeat(weight, NG, axis=0)
    perm = jnp.tile(perms, (nch_out, nch_in)).reshape(-1, nch_in, NNGB)
    wperm = jnp.take_along_axis(wrep, perm, axis=2).reshape(-1, nch_in * NNGB)
    brep = jnp.repeat(bias, NG, axis=0)
    return wperm, brep


def kernel(InStates, GnnPerms, gdiags, NNSites, SitesToShells, Psi_0, bias_0,
           Psi_1, bias_1, wtVC, ShellWeights):
    nb, _, ns = InStates.shape
    nsp = _NSP

    gw0, gb0 = _rotate_weight(Psi_0, bias_0, GnnPerms)
    gw1, gb1 = _rotate_weight(Psi_1, bias_1, GnnPerms)
    wtvc_rep = jnp.tile(wtVC, (NG, 1))
    perm2 = jnp.repeat(GnnPerms, DIM, axis=0)
    wtvc_transf = jnp.matmul(gdiags, jnp.take_along_axis(wtvc_rep, perm2,
                                                         axis=1))
    w2 = wtvc_transf.reshape(NG, DIM, NNGB).mean(axis=0)

    idx2d = jnp.pad(NNSites.astype(jnp.int32), ((0, 0), (0, nsp - ns)))
    x0p = jnp.pad(InStates[:, 0, :], ((0, 0), (0, nsp - ns)))
    sw = jnp.zeros((ns,), jnp.float32)
    for q in range(NSHELL):
        sw = jnp.where(SitesToShells == q, ShellWeights[q], sw)
    sw_site = jnp.pad(sw, (0, nsp - ns))

    t0 = _sc_table(x0p, NB, 2048)
    a = _sc_gather_t(t0, idx2d, 640)
    h0 = _tc_layer_a(a, gw0, gb0, 640, 4)

    t1 = _sc_table(h0, NCH0 * NB, 1024)
    bgat = _sc_gather_t(t1, idx2d, 640)
    h1 = _tc_layer_b(bgat, gw1, gb1, 640, 4)

    dp = _sc_hist(idx2d, sw_site, 2048)
    out = _tc_final(h1, dp, w2)
    return out * (1.0 / ns)


# speedup vs baseline: 3.8793x; 1.0093x over previous
import functools

import jax
import jax.numpy as jnp
from jax import lax
from jax.experimental import pallas as pl
from jax.experimental.pallas import tpu as pltpu
from jax.experimental.pallas import tpu_sc as plsc

NG = 48
NNGB = 12
DIM = 3
NSHELL = 6
NCH0 = 4
NB = 8

_NSP = 102400
_S = 2048
_NW = 32

_SC_PARAMS = pltpu.CompilerParams(use_tc_tiling_on_sc=False,
                                  needs_layout_passes=False)


def _mesh():
    return plsc.VectorSubcoreMesh(core_axis_name="core",
                                  subcore_axis_name="subcore")


def _wid():
    return lax.axis_index("subcore") * 2 + lax.axis_index("core")


def _iota16():
    return lax.iota(jnp.int32, 16)


def _sc_table(x, d, k):
    nsp = x.shape[1]
    nq = nsp // k

    @functools.partial(
        pl.kernel,
        out_type=jax.ShapeDtypeStruct((nsp, d), jnp.float32),
        mesh=_mesh(),
        scratch_types=[
            pltpu.VMEM((d, k), jnp.float32),
            pltpu.VMEM((k, d), jnp.float32),
            pltpu.SemaphoreType.DMA,
        ],
        compiler_params=_SC_PARAMS,
    )
    def kk(x_hbm, t_hbm, bufT, buf, sem):
        w = _wid()
        iot = _iota16()

        @pl.loop(0, (nq + _NW - 1) // _NW)
        def _(it):
            q = w + it * _NW

            @pl.when(q < nq)
            def _():
                s0 = q * k
                pltpu.async_copy(x_hbm.at[:, pl.ds(s0, k)], bufT, sem).wait()

                @pl.loop(0, k // 16)
                def _(g):
                    rows = g * 16 + iot
                    for m in range(d):
                        v = bufT[m, pl.ds(g * 16, 16)]
                        plsc.store_scatter(
                            buf, [rows, jnp.full((16,), m, jnp.int32)], v)

                pltpu.async_copy(buf, t_hbm.at[pl.ds(s0, k), :], sem).wait()

    return kk(x)


def _sc_gather_t(table, idx2d, k):
    nsp, d = table.shape
    nq = nsp // k
    nt = nq // _NW
    assert nt * _NW == nq
    sub = k // 128

    @functools.partial(
        pl.kernel,
        out_type=jax.ShapeDtypeStruct((NNGB, nq, d, k), jnp.float32),
        mesh=_mesh(),
        scratch_types=[
            pltpu.VMEM((2, k), jnp.int32),
            pltpu.VMEM((2, k, d), jnp.float32),
            pltpu.VMEM((2, d, k), jnp.float32),
            pltpu.SemaphoreType.DMA,
            pltpu.SemaphoreType.DMA,
            pltpu.SemaphoreType.DMA,
        ],
        compiler_params=_SC_PARAMS,
    )
    def kk(t_hbm, i_hbm, o_hbm, idxv, buf, bufT, semI, semG, semO):
        w = _wid()
        iot = _iota16()

        def fire_idx(j, t, p):
            q = w + t * _NW
            return pltpu.async_copy(i_hbm.at[j, pl.ds(q * k, k)],
                                    idxv.at[p], semI)

        def fire_gathers(p):
            return [
                pltpu.async_copy(
                    t_hbm.at[idxv.at[p].at[pl.ds(g * 128, 128)]],
                    buf.at[p].at[pl.ds(g * 128, 128), :], semG)
                for g in range(sub)
            ]

        def transpose(p):
            @pl.loop(0, k // 16)
            def _(g):
                rows = g * 16 + iot
                for m in range(d):
                    v = plsc.load_gather(
                        buf.at[p], [rows, jnp.full((16,), m, jnp.int32)])
                    bufT[p, m, pl.ds(g * 16, 16)] = v

        def fire_out(j, t, p):
            q = w + t * _NW
            return pltpu.async_copy(bufT.at[p], o_hbm.at[j, q], semO)

        G = 4
        U = G * nt

        @pl.loop(0, NNGB // G)
        def _(jj):
            j0 = jj * G
            jt = [(j0 + u // nt, u % nt) for u in range(U)]
            dI = {0: fire_idx(jt[0][0], jt[0][1], 0)}
            dG = {}
            dO = {}
            for u in range(U):
                p = u % 2
                dI[u].wait()
                dG[u] = fire_gathers(p)
                if u == 0:
                    if U > 1:
                        dI[1] = fire_idx(jt[1][0], jt[1][1], 1)
                else:
                    for c in dG[u - 1]:
                        c.wait()
                    if u + 1 < U:
                        dI[u + 1] = fire_idx(jt[u + 1][0], jt[u + 1][1],
                                             1 - p)
                    if u >= 3:
                        dO[u - 3].wait()
                    transpose(1 - p)
                    dO[u - 1] = fire_out(jt[u - 1][0], jt[u - 1][1], 1 - p)
            pl_ = (U - 1) % 2
            for c in dG[U - 1]:
                c.wait()
            if U >= 3:
                dO[U - 3].wait()
            transpose(pl_)
            dO[U - 1] = fire_out(jt[U - 1][0], jt[U - 1][1], pl_)
            if U >= 2:
                dO[U - 2].wait()
            dO[U - 1].wait()

    return kk(table, idx2d)


def _sc_hist(idx2d, sw_site, k):
    nsp = idx2d.shape[1]
    nq = nsp // k
    sub = k // 128
    zk = 1600
    per_tile = nsp // 16

    @functools.partial(
        pl.kernel,
        out_type=jax.ShapeDtypeStruct((2, NNGB, nsp), jnp.float32),
        mesh=_mesh(),
        scratch_types=[
            pltpu.VMEM((k,), jnp.int32),
            pltpu.VMEM((k,), jnp.float32),
            pltpu.VMEM((zk,), jnp.float32),
            pltpu.SemaphoreType.DMA,
        ] + [pltpu.VMEM_SHARED((nsp,), jnp.float32) for _ in range(NNGB)],
        compiler_params=_SC_PARAMS,
    )
    def kk(i_hbm, w_hbm, o_hbm, idxv, swv, zbuf, sem, *sh):
        cid = lax.axis_index("core")
        sid = lax.axis_index("subcore")
        w = _wid()

        @pl.loop(0, zk // 16)
        def _(z):
            zbuf[pl.ds(z * 16, 16)] = jnp.zeros((16,), jnp.float32)

        for j in range(NNGB):
            for r in range(per_tile // zk):
                pltpu.async_copy(
                    zbuf, sh[j].at[pl.ds(sid * per_tile + r * zk, zk)],
                    sem).wait()
        plsc.subcore_barrier()

        for j in range(NNGB):
            @pl.loop(0, (nq + _NW - 1) // _NW)
            def _(it):
                q = w + it * _NW

                @pl.when(q < nq)
                def _():
                    s0 = q * k
                    pltpu.async_copy(i_hbm.at[j, pl.ds(s0, k)], idxv,
                                     sem).wait()
                    pltpu.async_copy(w_hbm.at[pl.ds(s0, k)], swv, sem).wait()
                    cps = [
                        pltpu.async_copy(
                            swv.at[pl.ds(g * 128, 128)],
                            sh[j].at[idxv.at[pl.ds(g * 128, 128)]],
                            sem, add=True)
                        for g in range(sub)
                    ]
                    for c in cps:
                        c.wait()

        plsc.subcore_barrier()
        for j in range(NNGB):
            pltpu.async_copy(
                sh[j].at[pl.ds(sid * per_tile, per_tile)],
                o_hbm.at[cid, j, pl.ds(sid * per_tile, per_tile)],
                sem).wait()

    return kk(idx2d, sw_site)


def _softplus(x):
    return jnp.maximum(x, 0.0) + jnp.log(1.0 + jnp.exp(-jnp.abs(x)))


def _tc_layer_a(a, w, b, k, cpb):
    _, nq, _, _ = a.shape
    nsp = nq * k

    def body(a_ref, w_ref, b_ref, o_ref):
        wv = w_ref[...]
        bv = b_ref[...]
        for cc in range(cpb):
            for bb in range(NB):
                x = a_ref[:, cc, bb, :]
                h = jnp.dot(wv, x, preferred_element_type=jnp.float32) + bv
                h = _softplus(h)
                h = h.reshape(NCH0, NG, k).sum(axis=1) * (1.0 / NG)
                o_ref[4 * bb:4 * bb + 4, pl.ds(cc * k, k)] = h

    return pl.pallas_call(
        body,
        grid=(nq // cpb,),
        in_specs=[
            pl.BlockSpec((NNGB, cpb, NB, k), lambda i: (0, i, 0, 0)),
            pl.BlockSpec(w.shape, lambda i: (0, 0)),
            pl.BlockSpec(b.shape, lambda i: (0, 0)),
        ],
        out_specs=pl.BlockSpec((NCH0 * NB, cpb * k), lambda i: (0, i)),
        out_shape=jax.ShapeDtypeStruct((NCH0 * NB, nsp), jnp.float32),
    )(a, w, b)


def _tc_layer_b(bgat, w, b, k, cpb):
    _, nq, _, _ = bgat.shape
    nsp = nq * k

    def body(g_ref, w_ref, b_ref, o_ref):
        wv = w_ref[...]
        bv = b_ref[...]
        for cc in range(cpb):
            for bb in range(NB):
                h = bv
                for c in range(NCH0):
                    x = g_ref[:, cc, 4 * bb + c, :]
                    h = h + jnp.dot(wv[:, c * NNGB:(c + 1) * NNGB], x,
                                    preferred_element_type=jnp.float32)
                h = _softplus(h)
                o_ref[bb, pl.ds(cc * k, k)] = h.sum(axis=0) * (1.0 / NG)

    return pl.pallas_call(
        body,
        grid=(nq // cpb,),
        in_specs=[
            pl.BlockSpec((NNGB, cpb, NCH0 * NB, k), lambda i: (0, i, 0, 0)),
            pl.BlockSpec(w.shape, lambda i: (0, 0)),
            pl.BlockSpec(b.shape, lambda i: (0, 0)),
        ],
        out_specs=pl.BlockSpec((NB, cpb * k), lambda i: (0, i)),
        out_shape=jax.ShapeDtypeStruct((NB, nsp), jnp.float32),
    )(bgat, w, b)


def _tc_final(h1, dp, w2):
    _, nsp = h1.shape

    def body(h_ref, d_ref, w_ref, o_ref):
        i = pl.program_id(0)
        dsum = d_ref[0] + d_ref[1]
        c3 = jnp.dot(w_ref[...], dsum,
                     preferred_element_type=jnp.float32)
        p = lax.dot_general(h_ref[...], c3, (((1,), (1,)), ((), ())),
                            preferred_element_type=jnp.float32)

        @pl.when(i == 0)
        def _():
            o_ref[...] = jnp.zeros_like(o_ref)

        o_ref[...] += p

    return pl.pallas_call(
        body,
        grid=(nsp // _S,),
        in_specs=[
            pl.BlockSpec((NB, _S), lambda i: (0, i)),
            pl.BlockSpec((2, NNGB, _S), lambda i: (0, 0, i)),
            pl.BlockSpec(w2.shape, lambda i: (0, 0)),
        ],
        out_specs=pl.BlockSpec((NB, DIM), lambda i: (0, 0)),
        out_shape=jax.ShapeDtypeStruct((NB, DIM), jnp.float32),
    )(h1, dp, w2)


def _rotate_weight(weight, bias, perms):
    nch_out, nch_in, _ = weight.shape
    wrep = jnp.repeat(weight, NG, axis=0)
    perm = jnp.tile(perms, (nch_out, nch_in)).reshape(-1, nch_in, NNGB)
    wperm = jnp.take_along_axis(wrep, perm, axis=2).reshape(-1, nch_in * NNGB)
    brep = jnp.repeat(bias, NG, axis=0)
    return wperm, brep


def kernel(InStates, GnnPerms, gdiags, NNSites, SitesToShells, Psi_0, bias_0,
           Psi_1, bias_1, wtVC, ShellWeights):
    nb, _, ns = InStates.shape
    nsp = _NSP

    gw0, gb0 = _rotate_weight(Psi_0, bias_0, GnnPerms)
    gw1, gb1 = _rotate_weight(Psi_1, bias_1, GnnPerms)
    wtvc_rep = jnp.tile(wtVC, (NG, 1))
    perm2 = jnp.repeat(GnnPerms, DIM, axis=0)
    wtvc_transf = jnp.matmul(gdiags, jnp.take_along_axis(wtvc_rep, perm2,
                                                         axis=1))
    w2 = wtvc_transf.reshape(NG, DIM, NNGB).mean(axis=0)

    idx2d = jnp.pad(NNSites.astype(jnp.int32), ((0, 0), (0, nsp - ns)))
    x0p = jnp.pad(InStates[:, 0, :], ((0, 0), (0, nsp - ns)))
    sw = jnp.zeros((ns,), jnp.float32)
    for q in range(NSHELL):
        sw = jnp.where(SitesToShells == q, ShellWeights[q], sw)
    sw_site = jnp.pad(sw, (0, nsp - ns))

    t0 = _sc_table(x0p, NB, 2048)
    a = _sc_gather_t(t0, idx2d, 640)
    h0 = _tc_layer_a(a, gw0, gb0, 640, 4)

    t1 = _sc_table(h0, NCH0 * NB, 1024)
    bgat = _sc_gather_t(t1, idx2d, 640)
    h1 = _tc_layer_b(bgat, gw1, gb1, 640, 4)

    dp = _sc_hist(idx2d, sw_site, 2048)
    out = _tc_final(h1, dp, w2)
    return out * (1.0 / ns)


# speedup vs baseline: 4.0664x; 1.0482x over previous
import functools

import jax
import jax.numpy as jnp
from jax import lax
from jax.experimental import pallas as pl
from jax.experimental.pallas import tpu as pltpu
from jax.experimental.pallas import tpu_sc as plsc

NG = 48
NNGB = 12
DIM = 3
NSHELL = 6
NCH0 = 4
NB = 8

_NSP = 102400
_S = 2048
_NW = 32

_SC_PARAMS = pltpu.CompilerParams(use_tc_tiling_on_sc=False,
                                  needs_layout_passes=False)


def _mesh():
    return plsc.VectorSubcoreMesh(core_axis_name="core",
                                  subcore_axis_name="subcore")


def _wid():
    return lax.axis_index("subcore") * 2 + lax.axis_index("core")


def _iota16():
    return lax.iota(jnp.int32, 16)


def _sc_table(x, d, k):
    nsp = x.shape[1]
    nq = nsp // k

    @functools.partial(
        pl.kernel,
        out_type=jax.ShapeDtypeStruct((nsp, d), jnp.float32),
        mesh=_mesh(),
        scratch_types=[
            pltpu.VMEM((d, k), jnp.float32),
            pltpu.VMEM((k, d), jnp.float32),
            pltpu.SemaphoreType.DMA,
        ],
        compiler_params=_SC_PARAMS,
    )
    def kk(x_hbm, t_hbm, bufT, buf, sem):
        w = _wid()
        iot = _iota16()

        @pl.loop(0, (nq + _NW - 1) // _NW)
        def _(it):
            q = w + it * _NW

            @pl.when(q < nq)
            def _():
                s0 = q * k
                pltpu.async_copy(x_hbm.at[:, pl.ds(s0, k)], bufT, sem).wait()

                @pl.loop(0, k // 16)
                def _(g):
                    rows = g * 16 + iot
                    for m in range(d):
                        v = bufT[m, pl.ds(g * 16, 16)]
                        plsc.store_scatter(
                            buf, [rows, jnp.full((16,), m, jnp.int32)], v)

                pltpu.async_copy(buf, t_hbm.at[pl.ds(s0, k), :], sem).wait()

    return kk(x)


def _sc_gather_t(table, idx2d, k):
    nsp, d = table.shape
    nq = nsp // k
    nt = nq // _NW
    assert nt * _NW == nq
    sub = k // 128

    @functools.partial(
        pl.kernel,
        out_type=jax.ShapeDtypeStruct((NNGB, nq, d, k), jnp.float32),
        mesh=_mesh(),
        scratch_types=[
            pltpu.VMEM((2, k), jnp.int32),
            pltpu.VMEM((2, k, d), jnp.float32),
            pltpu.VMEM((2, d, k), jnp.float32),
            pltpu.SemaphoreType.DMA,
            pltpu.SemaphoreType.DMA,
            pltpu.SemaphoreType.DMA,
        ],
        compiler_params=_SC_PARAMS,
    )
    def kk(t_hbm, i_hbm, o_hbm, idxv, buf, bufT, semI, semG, semO):
        w = _wid()
        iot = _iota16()

        def fire_idx(j, t, p):
            q = w + t * _NW
            return pltpu.async_copy(i_hbm.at[j, pl.ds(q * k, k)],
                                    idxv.at[p], semI)

        def fire_gathers(p):
            return [
                pltpu.async_copy(
                    t_hbm.at[idxv.at[p].at[pl.ds(g * 128, 128)]],
                    buf.at[p].at[pl.ds(g * 128, 128), :], semG)
                for g in range(sub)
            ]

        def transpose(p):
            @pl.loop(0, k // 16)
            def _(g):
                rows = g * 16 + iot
                for m in range(d):
                    v = plsc.load_gather(
                        buf.at[p], [rows, jnp.full((16,), m, jnp.int32)])
                    bufT[p, m, pl.ds(g * 16, 16)] = v

        def fire_out(j, t, p):
            q = w + t * _NW
            return pltpu.async_copy(bufT.at[p], o_hbm.at[j, q], semO)

        G = 4
        U = G * nt

        @pl.loop(0, NNGB // G)
        def _(jj):
            j0 = jj * G
            jt = [(j0 + u // nt, u % nt) for u in range(U)]
            dI = {0: fire_idx(jt[0][0], jt[0][1], 0)}
            dG = {}
            dO = {}
            for u in range(U):
                p = u % 2
                dI[u].wait()
                dG[u] = fire_gathers(p)
                if u == 0:
                    if U > 1:
                        dI[1] = fire_idx(jt[1][0], jt[1][1], 1)
                else:
                    for c in dG[u - 1]:
                        c.wait()
                    if u + 1 < U:
                        dI[u + 1] = fire_idx(jt[u + 1][0], jt[u + 1][1],
                                             1 - p)
                    if u >= 3:
                        dO[u - 3].wait()
                    transpose(1 - p)
                    dO[u - 1] = fire_out(jt[u - 1][0], jt[u - 1][1], 1 - p)
            pl_ = (U - 1) % 2
            for c in dG[U - 1]:
                c.wait()
            if U >= 3:
                dO[U - 3].wait()
            transpose(pl_)
            dO[U - 1] = fire_out(jt[U - 1][0], jt[U - 1][1], pl_)
            if U >= 2:
                dO[U - 2].wait()
            dO[U - 1].wait()

    return kk(table, idx2d)


def _sc_stage_a(x, idx2d, k):
    d, nsp = x.shape
    nq = nsp // k
    nt = nq // _NW
    assert nt * _NW == nq
    sub = k // 128
    kt = 1280
    per_sub = nsp // 16

    @functools.partial(
        pl.kernel,
        out_type=jax.ShapeDtypeStruct((NNGB, nq, d, k), jnp.float32),
        mesh=_mesh(),
        scratch_types=[
            pltpu.VMEM((d, kt), jnp.float32),
            pltpu.VMEM((kt, d), jnp.float32),
            pltpu.VMEM((2, k), jnp.int32),
            pltpu.VMEM((2, k, d), jnp.float32),
            pltpu.VMEM((2, d, k), jnp.float32),
            pltpu.SemaphoreType.DMA,
            pltpu.SemaphoreType.DMA,
            pltpu.SemaphoreType.DMA,
            pltpu.SemaphoreType.DMA,
            pltpu.VMEM_SHARED((nsp, d), jnp.float32),
        ],
        compiler_params=_SC_PARAMS,
    )
    def kk(x_hbm, i_hbm, o_hbm, bufX, bufS, idxv, buf, bufT,
           semT, semI, semG, semO, sh):
        w = _wid()
        sid = lax.axis_index("subcore")
        iot = _iota16()

        @pl.loop(0, per_sub // kt)
        def _(r):
            s0 = sid * per_sub + r * kt
            pltpu.async_copy(x_hbm.at[:, pl.ds(s0, kt)], bufX, semT).wait()

            @pl.loop(0, kt // 16)
            def _(g):
                rows = g * 16 + iot
                for m in range(d):
                    v = bufX[m, pl.ds(g * 16, 16)]
                    plsc.store_scatter(
                        bufS, [rows, jnp.full((16,), m, jnp.int32)], v)

            pltpu.async_copy(bufS, sh.at[pl.ds(s0, kt), :], semT).wait()

        plsc.subcore_barrier()

        def fire_idx(j, t, p):
            q = w + t * _NW
            return pltpu.async_copy(i_hbm.at[j, pl.ds(q * k, k)],
                                    idxv.at[p], semI)

        def fire_gathers(p):
            return [
                pltpu.async_copy(
                    sh.at[idxv.at[p].at[pl.ds(g * 128, 128)]],
                    buf.at[p].at[pl.ds(g * 128, 128), :], semG)
                for g in range(sub)
            ]

        def transpose(p):
            @pl.loop(0, k // 16)
            def _(g):
                rows = g * 16 + iot
                for m in range(d):
                    v = plsc.load_gather(
                        buf.at[p], [rows, jnp.full((16,), m, jnp.int32)])
                    bufT[p, m, pl.ds(g * 16, 16)] = v

        def fire_out(j, t, p):
            q = w + t * _NW
            return pltpu.async_copy(bufT.at[p], o_hbm.at[j, q], semO)

        G = 4
        U = G * nt

        @pl.loop(0, NNGB // G)
        def _(jj):
            j0 = jj * G
            jt = [(j0 + u // nt, u % nt) for u in range(U)]
            dI = {0: fire_idx(jt[0][0], jt[0][1], 0)}
            dG = {}
            dO = {}
            for u in range(U):
                p = u % 2
                dI[u].wait()
                dG[u] = fire_gathers(p)
                if u == 0:
                    if U > 1:
                        dI[1] = fire_idx(jt[1][0], jt[1][1], 1)
                else:
                    for c in dG[u - 1]:
                        c.wait()
                    if u + 1 < U:
                        dI[u + 1] = fire_idx(jt[u + 1][0], jt[u + 1][1],
                                             1 - p)
                    if u >= 3:
                        dO[u - 3].wait()
                    transpose(1 - p)
                    dO[u - 1] = fire_out(jt[u - 1][0], jt[u - 1][1], 1 - p)
            pl_ = (U - 1) % 2
            for c in dG[U - 1]:
                c.wait()
            if U >= 3:
                dO[U - 3].wait()
            transpose(pl_)
            dO[U - 1] = fire_out(jt[U - 1][0], jt[U - 1][1], pl_)
            if U >= 2:
                dO[U - 2].wait()
            dO[U - 1].wait()

    return kk(x, idx2d)


def _sc_hist(idx2d, sw_site, k):
    nsp = idx2d.shape[1]
    nq = nsp // k
    sub = k // 128
    zk = 1600
    per_tile = nsp // 16

    @functools.partial(
        pl.kernel,
        out_type=jax.ShapeDtypeStruct((2, NNGB, nsp), jnp.float32),
        mesh=_mesh(),
        scratch_types=[
            pltpu.VMEM((k,), jnp.int32),
            pltpu.VMEM((k,), jnp.float32),
            pltpu.VMEM((zk,), jnp.float32),
            pltpu.SemaphoreType.DMA,
        ] + [pltpu.VMEM_SHARED((nsp,), jnp.float32) for _ in range(NNGB)],
        compiler_params=_SC_PARAMS,
    )
    def kk(i_hbm, w_hbm, o_hbm, idxv, swv, zbuf, sem, *sh):
        cid = lax.axis_index("core")
        sid = lax.axis_index("subcore")
        w = _wid()

        @pl.loop(0, zk // 16)
        def _(z):
            zbuf[pl.ds(z * 16, 16)] = jnp.zeros((16,), jnp.float32)

        for j in range(NNGB):
            for r in range(per_tile // zk):
                pltpu.async_copy(
                    zbuf, sh[j].at[pl.ds(sid * per_tile + r * zk, zk)],
                    sem).wait()
        plsc.subcore_barrier()

        for j in range(NNGB):
            @pl.loop(0, (nq + _NW - 1) // _NW)
            def _(it):
                q = w + it * _NW

                @pl.when(q < nq)
                def _():
                    s0 = q * k
                    pltpu.async_copy(i_hbm.at[j, pl.ds(s0, k)], idxv,
                                     sem).wait()
                    pltpu.async_copy(w_hbm.at[pl.ds(s0, k)], swv, sem).wait()
                    cps = [
                        pltpu.async_copy(
                            swv.at[pl.ds(g * 128, 128)],
                            sh[j].at[idxv.at[pl.ds(g * 128, 128)]],
                            sem, add=True)
                        for g in range(sub)
                    ]
                    for c in cps:
                        c.wait()

        plsc.subcore_barrier()
        for j in range(NNGB):
            pltpu.async_copy(
                sh[j].at[pl.ds(sid * per_tile, per_tile)],
                o_hbm.at[cid, j, pl.ds(sid * per_tile, per_tile)],
                sem).wait()

    return kk(idx2d, sw_site)


def _softplus(x):
    return jnp.maximum(x, 0.0) + jnp.log(1.0 + jnp.exp(-jnp.abs(x)))


def _tc_layer_a(a, w, b, k, cpb):
    _, nq, _, _ = a.shape
    nsp = nq * k

    def body(a_ref, w_ref, b_ref, o_ref):
        wv = w_ref[...]
        bv = b_ref[...]
        for cc in range(cpb):
            for bb in range(NB):
                x = a_ref[:, cc, bb, :]
                h = jnp.dot(wv, x, preferred_element_type=jnp.float32) + bv
                h = _softplus(h)
                h = h.reshape(NCH0, NG, k).sum(axis=1) * (1.0 / NG)
                o_ref[4 * bb:4 * bb + 4, pl.ds(cc * k, k)] = h

    return pl.pallas_call(
        body,
        grid=(nq // cpb,),
        in_specs=[
            pl.BlockSpec((NNGB, cpb, NB, k), lambda i: (0, i, 0, 0)),
            pl.BlockSpec(w.shape, lambda i: (0, 0)),
            pl.BlockSpec(b.shape, lambda i: (0, 0)),
        ],
        out_specs=pl.BlockSpec((NCH0 * NB, cpb * k), lambda i: (0, i)),
        out_shape=jax.ShapeDtypeStruct((NCH0 * NB, nsp), jnp.float32),
    )(a, w, b)


def _tc_layer_b(bgat, w, b, k, cpb):
    _, nq, _, _ = bgat.shape
    nsp = nq * k

    def body(g_ref, w_ref, b_ref, o_ref):
        wv = w_ref[...]
        bv = b_ref[...]
        for cc in range(cpb):
            for bb in range(NB):
                h = bv
                for c in range(NCH0):
                    x = g_ref[:, cc, 4 * bb + c, :]
                    h = h + jnp.dot(wv[:, c * NNGB:(c + 1) * NNGB], x,
                                    preferred_element_type=jnp.float32)
                h = _softplus(h)
                o_ref[bb, pl.ds(cc * k, k)] = h.sum(axis=0) * (1.0 / NG)

    return pl.pallas_call(
        body,
        grid=(nq // cpb,),
        in_specs=[
            pl.BlockSpec((NNGB, cpb, NCH0 * NB, k), lambda i: (0, i, 0, 0)),
            pl.BlockSpec(w.shape, lambda i: (0, 0)),
            pl.BlockSpec(b.shape, lambda i: (0, 0)),
        ],
        out_specs=pl.BlockSpec((NB, cpb * k), lambda i: (0, i)),
        out_shape=jax.ShapeDtypeStruct((NB, nsp), jnp.float32),
    )(bgat, w, b)


def _tc_final(h1, dp, w2):
    _, nsp = h1.shape

    def body(h_ref, d_ref, w_ref, o_ref):
        i = pl.program_id(0)
        dsum = d_ref[0] + d_ref[1]
        c3 = jnp.dot(w_ref[...], dsum,
                     preferred_element_type=jnp.float32)
        p = lax.dot_general(h_ref[...], c3, (((1,), (1,)), ((), ())),
                            preferred_element_type=jnp.float32)

        @pl.when(i == 0)
        def _():
            o_ref[...] = jnp.zeros_like(o_ref)

        o_ref[...] += p

    return pl.pallas_call(
        body,
        grid=(nsp // _S,),
        in_specs=[
            pl.BlockSpec((NB, _S), lambda i: (0, i)),
            pl.BlockSpec((2, NNGB, _S), lambda i: (0, 0, i)),
            pl.BlockSpec(w2.shape, lambda i: (0, 0)),
        ],
        out_specs=pl.BlockSpec((NB, DIM), lambda i: (0, 0)),
        out_shape=jax.ShapeDtypeStruct((NB, DIM), jnp.float32),
    )(h1, dp, w2)


def _rotate_weight(weight, bias, perms):
    nch_out, nch_in, _ = weight.shape
    wrep = jnp.repeat(weight, NG, axis=0)
    perm = jnp.tile(perms, (nch_out, nch_in)).reshape(-1, nch_in, NNGB)
    wperm = jnp.take_along_axis(wrep, perm, axis=2).reshape(-1, nch_in * NNGB)
    brep = jnp.repeat(bias, NG, axis=0)
    return wperm, brep


def kernel(InStates, GnnPerms, gdiags, NNSites, SitesToShells, Psi_0, bias_0,
           Psi_1, bias_1, wtVC, ShellWeights):
    nb, _, ns = InStates.shape
    nsp = _NSP

    gw0, gb0 = _rotate_weight(Psi_0, bias_0, GnnPerms)
    gw1, gb1 = _rotate_weight(Psi_1, bias_1, GnnPerms)
    wtvc_rep = jnp.tile(wtVC, (NG, 1))
    perm2 = jnp.repeat(GnnPerms, DIM, axis=0)
    wtvc_transf = jnp.matmul(gdiags, jnp.take_along_axis(wtvc_rep, perm2,
                                                         axis=1))
    w2 = wtvc_transf.reshape(NG, DIM, NNGB).mean(axis=0)

    idx2d = jnp.pad(NNSites.astype(jnp.int32), ((0, 0), (0, nsp - ns)))
    x0p = jnp.pad(InStates[:, 0, :], ((0, 0), (0, nsp - ns)))
    sw = jnp.zeros((ns,), jnp.float32)
    for q in range(NSHELL):
        sw = jnp.where(SitesToShells == q, ShellWeights[q], sw)
    sw_site = jnp.pad(sw, (0, nsp - ns))

    a = _sc_stage_a(x0p, idx2d, 640)
    h0 = _tc_layer_a(a, gw0, gb0, 640, 4)

    t1 = _sc_table(h0, NCH0 * NB, 1024)
    bgat = _sc_gather_t(t1, idx2d, 640)
    h1 = _tc_layer_b(bgat, gw1, gb1, 640, 4)

    dp = _sc_hist(idx2d, sw_site, 2048)
    out = _tc_final(h1, dp, w2)
    return out * (1.0 / ns)
